# Initial kernel scaffold; baseline (speedup 1.0000x reference)
import jax
import jax.numpy as jnp
from jax.experimental import pallas as pl


def kernel(x):
    raise NotImplementedError("write your pallas kernel here")


# speedup vs baseline: 4.0433x; 4.0433x over previous
import jax
import jax.numpy as jnp
from jax.experimental import pallas as pl
from jax.experimental.pallas import tpu as pltpu

S = 128
SCALE = float(S // 2 - 2)


def _splat_kernel(x_ref, o_ref, c0_ref, c1_ref, w_ref):
    R = x_ref.shape[2]

    for b in range(2):
        pc = x_ref[b] * SCALE
        clf = jnp.trunc(pc)
        cli = clf.astype(jnp.int32)
        feat = 2.0 - jnp.abs(clf - pc).sum(axis=0)
        c0 = cli[0] - jnp.min(cli[0])
        c1 = cli[1] - jnp.min(cli[1])
        oob = (c0 >= S) | (c1 >= S)
        c0 = jnp.where(c0 >= S, 0, c0)
        c1 = jnp.where(c1 >= S, 0, c1)
        w = jnp.where(oob, 0.0, feat)
        c0_ref[b] = c0
        c1_ref[b] = c1 + b * S
        w_ref[b] = w

    iota_a = jax.lax.broadcasted_iota(jnp.int32, (S, 2 * S), 0)
    iota_c = jax.lax.broadcasted_iota(jnp.int32, (2 * S, 2 * S), 0)

    def body(i, acc):
        base = i * 8
        t0a = c0_ref[0, pl.ds(base, 8), :]
        t0b = c0_ref[1, pl.ds(base, 8), :]
        t1a = c1_ref[0, pl.ds(base, 8), :]
        t1b = c1_ref[1, pl.ds(base, 8), :]
        twa = w_ref[0, pl.ds(base, 8), :]
        twb = w_ref[1, pl.ds(base, 8), :]
        for u in range(8):
            r_cat = jnp.concatenate([t0a[u:u + 1, :], t0b[u:u + 1, :]], axis=1)
            c_cat = jnp.concatenate([t1a[u:u + 1, :], t1b[u:u + 1, :]], axis=1)
            w_cat = jnp.concatenate([twa[u:u + 1, :], twb[u:u + 1, :]], axis=1)
            a_mat = jnp.where(r_cat == iota_a, w_cat, 0.0)
            c_mat = jnp.where(c_cat == iota_c, 1.0, 0.0)
            acc = acc + jax.lax.dot_general(
                a_mat, c_mat, (((1,), (1,)), ((), ())),
                preferred_element_type=jnp.float32)
        return acc

    acc = jax.lax.fori_loop(0, R // 8, body,
                            jnp.zeros((S, 2 * S), jnp.float32))
    o_ref[0] = acc[:, :S]
    o_ref[1] = acc[:, S:]


def kernel(x):
    B, C, N = x.shape
    R = N // 128
    xr = x.reshape(B, C, R, 128)
    out = pl.pallas_call(
        _splat_kernel,
        grid=(B // 2,),
        in_specs=[pl.BlockSpec((2, 2, R, 128), lambda p: (p, 0, 0, 0))],
        out_specs=pl.BlockSpec((2, S, S), lambda p: (p, 0, 0)),
        out_shape=jax.ShapeDtypeStruct((B, S, S), jnp.float32),
        scratch_shapes=[
            pltpu.VMEM((2, R, 128), jnp.int32),
            pltpu.VMEM((2, R, 128), jnp.int32),
            pltpu.VMEM((2, R, 128), jnp.float32),
        ],
        compiler_params=pltpu.CompilerParams(
            dimension_semantics=("parallel",)),
    )(xr)
    return out[:, None, :, :]


# speedup vs baseline: 4.3009x; 1.0637x over previous
import jax
import jax.numpy as jnp
from jax.experimental import pallas as pl
from jax.experimental.pallas import tpu as pltpu

S = 128
SCALE = float(S // 2 - 2)


def _splat_kernel(x_ref, o_ref, c0_ref, c1_ref, w_ref):
    R = x_ref.shape[2]

    for b in range(2):
        pc = x_ref[b] * SCALE
        clf = jnp.trunc(pc)
        cli = clf.astype(jnp.int32)
        feat = 2.0 - jnp.abs(clf - pc).sum(axis=0)
        c0 = cli[0] - jnp.min(cli[0])
        c1 = cli[1] - jnp.min(cli[1])
        oob = (c0 >= S) | (c1 >= S)
        c0 = jnp.where(c0 >= S, 0, c0)
        c1 = jnp.where(c1 >= S, 0, c1)
        w = jnp.where(oob, 0.0, feat)
        c0_ref[b] = c0
        c1_ref[b] = c1 + b * S
        w_ref[b] = w

    iota_a = jax.lax.broadcasted_iota(jnp.int32, (S, 2 * S), 0)
    iota_c = jax.lax.broadcasted_iota(jnp.int32, (2 * S, 2 * S), 0)

    def body(i, acc):
        base = i * 8
        t0a = c0_ref[0, pl.ds(base, 8), :]
        t0b = c0_ref[1, pl.ds(base, 8), :]
        t1a = c1_ref[0, pl.ds(base, 8), :]
        t1b = c1_ref[1, pl.ds(base, 8), :]
        twa = w_ref[0, pl.ds(base, 8), :]
        twb = w_ref[1, pl.ds(base, 8), :]
        for u in range(8):
            r_cat = jnp.concatenate([t0a[u:u + 1, :], t0b[u:u + 1, :]], axis=1)
            c_cat = jnp.concatenate([t1a[u:u + 1, :], t1b[u:u + 1, :]], axis=1)
            w_cat = jnp.concatenate([twa[u:u + 1, :], twb[u:u + 1, :]], axis=1)
            a_mat = jnp.where(r_cat == iota_a, w_cat, 0.0)
            c_mat = jnp.where(c_cat == iota_c, 1.0, 0.0)
            acc = acc + jax.lax.dot_general(
                a_mat, c_mat, (((1,), (1,)), ((), ())),
                preferred_element_type=jnp.float32)
        return acc

    acc = jax.lax.fori_loop(0, R // 8, body,
                            jnp.zeros((S, 2 * S), jnp.float32))
    o_ref[0] = acc[:, :S]
    o_ref[1] = acc[:, S:]


def kernel(x):
    B, C, N = x.shape
    R = N // 128
    xr = jax.lax.slice(x, (0, 0, 0), (B, 2, N)).reshape(B, 2, R, 128)
    out = pl.pallas_call(
        _splat_kernel,
        grid=(B // 2,),
        in_specs=[pl.BlockSpec((2, 2, R, 128), lambda p: (p, 0, 0, 0))],
        out_specs=pl.BlockSpec((2, S, S), lambda p: (p, 0, 0)),
        out_shape=jax.ShapeDtypeStruct((B, S, S), jnp.float32),
        scratch_shapes=[
            pltpu.VMEM((2, R, 128), jnp.int32),
            pltpu.VMEM((2, R, 128), jnp.int32),
            pltpu.VMEM((2, R, 128), jnp.float32),
        ],
        compiler_params=pltpu.CompilerParams(
            dimension_semantics=("parallel",)),
    )(xr)
    return out[:, None, :, :]


# speedup vs baseline: 5.0378x; 1.1713x over previous
import jax
import jax.numpy as jnp
from jax.experimental import pallas as pl
from jax.experimental.pallas import tpu as pltpu

S = 128
SCALE = float(S // 2 - 2)


def _splat_kernel(x_ref, o_ref, cr_ref, cc_ref, w_ref):
    R = x_ref.shape[2]

    for b in range(2):
        pc = x_ref[b] * SCALE
        clf = jnp.trunc(pc)
        cli = clf.astype(jnp.int32)
        feat = 2.0 - jnp.abs(clf - pc).sum(axis=0)
        c0 = cli[0] - jnp.min(cli[0])
        c1 = cli[1] - jnp.min(cli[1])
        oob = (c0 >= S) | (c1 >= S)
        c0 = jnp.where(c0 >= S, 0, c0)
        c1 = jnp.where(c1 >= S, 0, c1)
        w = jnp.where(oob, 0.0, feat)
        cr_ref[:, b * S:(b + 1) * S] = c0.astype(jnp.int16)
        cc_ref[:, b * S:(b + 1) * S] = (c1 + b * S).astype(jnp.int16)
        w_ref[:, b * S:(b + 1) * S] = w.astype(jnp.bfloat16)

    iota_a = jax.lax.broadcasted_iota(jnp.int16, (S, 2 * S), 0)
    iota_c = jax.lax.broadcasted_iota(jnp.int16, (2 * S, 2 * S), 0)
    one = jnp.bfloat16(1.0)
    zero = jnp.bfloat16(0.0)

    def body(i, acc):
        base = i * 16
        tr = cr_ref[pl.ds(base, 16), :]
        tc = cc_ref[pl.ds(base, 16), :]
        tw = w_ref[pl.ds(base, 16), :]
        for u in range(16):
            r_row = tr[u:u + 1, :]
            c_row = tc[u:u + 1, :]
            w_row = tw[u:u + 1, :]
            a_mat = jnp.where(r_row == iota_a, w_row, zero)
            c_mat = jnp.where(c_row == iota_c, one, zero)
            acc = acc + jax.lax.dot_general(
                a_mat, c_mat, (((1,), (1,)), ((), ())),
                preferred_element_type=jnp.float32)
        return acc

    acc = jax.lax.fori_loop(0, R // 16, body,
                            jnp.zeros((S, 2 * S), jnp.float32))
    o_ref[0] = acc[:, :S]
    o_ref[1] = acc[:, S:]


def kernel(x):
    B, C, N = x.shape
    R = N // 128
    xr = jax.lax.slice(x, (0, 0, 0), (B, 2, N)).reshape(B, 2, R, 128)
    out = pl.pallas_call(
        _splat_kernel,
        grid=(B // 2,),
        in_specs=[pl.BlockSpec((2, 2, R, 128), lambda p: (p, 0, 0, 0))],
        out_specs=pl.BlockSpec((2, S, S), lambda p: (p, 0, 0)),
        out_shape=jax.ShapeDtypeStruct((B, S, S), jnp.float32),
        scratch_shapes=[
            pltpu.VMEM((R, 2 * S), jnp.int16),
            pltpu.VMEM((R, 2 * S), jnp.int16),
            pltpu.VMEM((R, 2 * S), jnp.bfloat16),
        ],
        compiler_params=pltpu.CompilerParams(
            dimension_semantics=("parallel",)),
    )(xr)
    return out[:, None, :, :]


# speedup vs baseline: 5.7144x; 1.1343x over previous
import jax
import jax.numpy as jnp
from jax.experimental import pallas as pl
from jax.experimental.pallas import tpu as pltpu

S = 128
SCALE = float(S // 2 - 2)


def _splat_kernel(x_ref, o_ref, cr_ref, cc_ref, w_ref):
    R = x_ref.shape[2]

    for b in range(2):
        pc = x_ref[b] * SCALE
        clf = jnp.trunc(pc)
        cli = clf.astype(jnp.int32)
        feat = 2.0 - jnp.abs(clf - pc).sum(axis=0)
        c0 = cli[0] - jnp.min(cli[0])
        c1 = cli[1] - jnp.min(cli[1])
        oob = (c0 >= S) | (c1 >= S)
        c0 = jnp.where(c0 >= S, 0, c0)
        c1 = jnp.where(c1 >= S, 0, c1)
        w = jnp.where(oob, 0.0, feat)
        cr_ref[:, b * S:(b + 1) * S] = c0.astype(jnp.int16)
        cc_ref[:, b * S:(b + 1) * S] = c1 + b * S
        w_ref[:, b * S:(b + 1) * S] = w.astype(jnp.bfloat16)

    iota_a = jax.lax.broadcasted_iota(jnp.int16, (S, 2 * S), 0)
    iota_ct = jax.lax.broadcasted_iota(jnp.int16, (2 * S, 2 * S), 1)
    one = jnp.bfloat16(1.0)
    zero = jnp.bfloat16(0.0)

    def body(i, acc):
        base = i * 16
        tr = cr_ref[pl.ds(base, 16), :]
        tc = cc_ref[pl.ds(base, 16), :]
        tw = w_ref[pl.ds(base, 16), :]
        tct = jnp.transpose(tc).astype(jnp.int16)
        for u in range(16):
            r_row = tr[u:u + 1, :]
            w_row = tw[u:u + 1, :]
            c_col = tct[:, u:u + 1]
            a_mat = jnp.where(r_row == iota_a, w_row, zero)
            ct_mat = jnp.where(c_col == iota_ct, one, zero)
            acc = acc + jnp.dot(a_mat, ct_mat,
                                preferred_element_type=jnp.float32)
        return acc

    acc = jax.lax.fori_loop(0, R // 16, body,
                            jnp.zeros((S, 2 * S), jnp.float32))
    o_ref[0] = acc[:, :S]
    o_ref[1] = acc[:, S:]


def kernel(x):
    B, C, N = x.shape
    R = N // 128
    xr = jax.lax.slice(x, (0, 0, 0), (B, 2, N)).reshape(B, 2, R, 128)
    out = pl.pallas_call(
        _splat_kernel,
        grid=(B // 2,),
        in_specs=[pl.BlockSpec((2, 2, R, 128), lambda p: (p, 0, 0, 0))],
        out_specs=pl.BlockSpec((2, S, S), lambda p: (p, 0, 0)),
        out_shape=jax.ShapeDtypeStruct((B, S, S), jnp.float32),
        scratch_shapes=[
            pltpu.VMEM((R, 2 * S), jnp.int16),
            pltpu.VMEM((R, 2 * S), jnp.int32),
            pltpu.VMEM((R, 2 * S), jnp.bfloat16),
        ],
        compiler_params=pltpu.CompilerParams(
            dimension_semantics=("parallel",)),
    )(xr)
    return out[:, None, :, :]


# speedup vs baseline: 6.5576x; 1.1476x over previous
import jax
import jax.numpy as jnp
from jax.experimental import pallas as pl
from jax.experimental.pallas import tpu as pltpu

S = 128
SCALE = float(S // 2 - 2)


def _splat_kernel(x_ref, o_ref, cr_ref, cc_ref, w_ref):
    R = x_ref.shape[2]

    for b in range(2):
        pc = x_ref[b] * SCALE
        clf = jnp.trunc(pc)
        cli = clf.astype(jnp.int32)
        feat = 2.0 - jnp.abs(clf - pc).sum(axis=0)
        c0 = cli[0] - jnp.min(cli[0])
        c1 = cli[1] - jnp.min(cli[1])
        oob = (c0 >= S) | (c1 >= S)
        c0 = jnp.where(c0 >= S, 0, c0)
        c1 = jnp.where(c1 >= S, 0, c1)
        w = jnp.where(oob, 0.0, feat)
        cr_ref[:, b * S:(b + 1) * S] = c0.astype(jnp.int16)
        cc_ref[:, b * S:(b + 1) * S] = c1 + b * S
        w_ref[:, b * S:(b + 1) * S] = w.astype(jnp.bfloat16)

    iota_a = jax.lax.broadcasted_iota(jnp.int16, (S, 2 * S), 0)
    iota_ct = jax.lax.broadcasted_iota(jnp.int32, (2 * S, 2 * S), 1).astype(jnp.uint8)
    one8 = jnp.float8_e4m3fn(1.0)
    zero8 = jnp.float8_e4m3fn(0.0)
    zero = jnp.bfloat16(0.0)

    def body(i, acc):
        base = i * 16
        tr = cr_ref[pl.ds(base, 16), :]
        tc = cc_ref[pl.ds(base, 16), :]
        tw = w_ref[pl.ds(base, 16), :]
        tct = jnp.transpose(tc).astype(jnp.uint8)
        for u in range(16):
            r_row = tr[u:u + 1, :]
            w_row = tw[u:u + 1, :]
            c_col = tct[:, u:u + 1]
            a_mat = jnp.where(r_row == iota_a, w_row, zero)
            ct_mat = jnp.where(c_col == iota_ct, one8, zero8)
            acc = acc + jax.lax.dot_general(
                a_mat, ct_mat, (((1,), (0,)), ((), ())),
                preferred_element_type=jnp.float32)
        return acc

    acc = jax.lax.fori_loop(0, R // 16, body,
                            jnp.zeros((S, 2 * S), jnp.float32))
    o_ref[0] = acc[:, :S]
    o_ref[1] = acc[:, S:]


def kernel(x):
    B, C, N = x.shape
    R = N // 128
    xr = jax.lax.slice(x, (0, 0, 0), (B, 2, N)).reshape(B, 2, R, 128)
    out = pl.pallas_call(
        _splat_kernel,
        grid=(B // 2,),
        in_specs=[pl.BlockSpec((2, 2, R, 128), lambda p: (p, 0, 0, 0))],
        out_specs=pl.BlockSpec((2, S, S), lambda p: (p, 0, 0)),
        out_shape=jax.ShapeDtypeStruct((B, S, S), jnp.float32),
        scratch_shapes=[
            pltpu.VMEM((R, 2 * S), jnp.int16),
            pltpu.VMEM((R, 2 * S), jnp.int32),
            pltpu.VMEM((R, 2 * S), jnp.bfloat16),
        ],
        compiler_params=pltpu.CompilerParams(
            dimension_semantics=("parallel",)),
    )(xr)
    return out[:, None, :, :]


# speedup vs baseline: 8.2827x; 1.2631x over previous
import jax
import jax.numpy as jnp
from jax.experimental import pallas as pl
from jax.experimental.pallas import tpu as pltpu

S = 128
SCALE = float(S // 2 - 2)


def _splat_kernel(x_ref, o_ref, cr_ref, cc_ref, w_ref):
    R = x_ref.shape[2]

    for b in range(2):
        pc = x_ref[b] * SCALE
        clf = jnp.trunc(pc)
        cli = clf.astype(jnp.int32)
        feat = 2.0 - jnp.abs(clf - pc).sum(axis=0)
        c0 = cli[0] - jnp.min(cli[0])
        c1 = cli[1] - jnp.min(cli[1])
        oob = (c0 >= S) | (c1 >= S)
        c0 = jnp.where(c0 >= S, 0, c0)
        c1 = jnp.where(c1 >= S, 0, c1)
        w = jnp.where(oob, 0.0, feat)
        cr_ref[:, b * S:(b + 1) * S] = c0.astype(jnp.int16)
        cc_ref[:, b * S:(b + 1) * S] = c1 + b * S
        w_ref[:, b * S:(b + 1) * S] = w.astype(jnp.bfloat16)

    iota_a = jax.lax.broadcasted_iota(jnp.int16, (S, 2 * S), 0)
    iota_ct = jax.lax.broadcasted_iota(jnp.int32, (2 * S, 2 * S), 1).astype(jnp.uint8)
    one8 = jnp.float8_e4m3fn(1.0)
    zero8 = jnp.float8_e4m3fn(0.0)
    zero = jnp.bfloat16(0.0)

    def body(i, acc):
        for h in range(2):
            base = i * 32 + h * 16
            tr = cr_ref[pl.ds(base, 16), :]
            tc = cc_ref[pl.ds(base, 16), :]
            tw = w_ref[pl.ds(base, 16), :]
            tct = jnp.transpose(tc).astype(jnp.uint8)
            for u in range(16):
                r_row = tr[u:u + 1, :]
                w_row = tw[u:u + 1, :]
                c_col = tct[:, u:u + 1]
                a_mat = jnp.where(r_row == iota_a, w_row, zero)
                ct_mat = jnp.where(c_col == iota_ct, one8, zero8)
                acc = acc + jax.lax.dot_general(
                    a_mat, ct_mat, (((1,), (0,)), ((), ())),
                    preferred_element_type=jnp.float32)
        return acc

    acc = jax.lax.fori_loop(0, R // 32, body,
                            jnp.zeros((S, 2 * S), jnp.float32))
    o_ref[0] = acc[:, :S]
    o_ref[1] = acc[:, S:]


def kernel(x):
    B, C, N = x.shape
    R = N // 128
    xr = jax.lax.slice(x, (0, 0, 0), (B, 2, N)).reshape(B, 2, R, 128)
    out = pl.pallas_call(
        _splat_kernel,
        grid=(B // 2,),
        in_specs=[pl.BlockSpec((2, 2, R, 128), lambda p: (p, 0, 0, 0))],
        out_specs=pl.BlockSpec((2, S, S), lambda p: (p, 0, 0)),
        out_shape=jax.ShapeDtypeStruct((B, S, S), jnp.float32),
        scratch_shapes=[
            pltpu.VMEM((R, 2 * S), jnp.int16),
            pltpu.VMEM((R, 2 * S), jnp.int32),
            pltpu.VMEM((R, 2 * S), jnp.bfloat16),
        ],
        compiler_params=pltpu.CompilerParams(
            dimension_semantics=("parallel",)),
    )(xr)
    return out[:, None, :, :]


# speedup vs baseline: 9.6158x; 1.1610x over previous
import jax
import jax.numpy as jnp
from jax.experimental import pallas as pl
from jax.experimental.pallas import tpu as pltpu

S = 128
SCALE = float(S // 2 - 2)


def _splat_kernel(x_ref, o_ref, cr_ref, cc_ref, w_ref):
    R = x_ref.shape[2]

    for b in range(2):
        pc = x_ref[b] * SCALE
        clf = jnp.trunc(pc)
        cli = clf.astype(jnp.int32)
        feat = 2.0 - jnp.abs(clf - pc).sum(axis=0)
        c0 = cli[0] - jnp.min(cli[0])
        c1 = cli[1] - jnp.min(cli[1])
        oob = (c0 >= S) | (c1 >= S)
        c0 = jnp.where(c0 >= S, 0, c0)
        c1 = jnp.where(c1 >= S, 0, c1)
        w = jnp.where(oob, 0.0, feat)
        cr_ref[:, b * S:(b + 1) * S] = c0.astype(jnp.int16)
        cc_ref[:, b * S:(b + 1) * S] = c1 + b * S
        w_ref[:, b * S:(b + 1) * S] = w.astype(jnp.bfloat16)

    iota_a = jax.lax.broadcasted_iota(jnp.int16, (S, 2 * S), 0)
    iota_ct = jax.lax.broadcasted_iota(jnp.int32, (2 * S, 2 * S), 1).astype(jnp.uint8)
    one8 = jnp.float8_e4m3fn(1.0)
    zero8 = jnp.float8_e4m3fn(0.0)
    zero = jnp.bfloat16(0.0)

    def body(i, acc):
        for h in range(4):
            base = i * 64 + h * 16
            tr = cr_ref[pl.ds(base, 16), :]
            tc = cc_ref[pl.ds(base, 16), :]
            tw = w_ref[pl.ds(base, 16), :]
            tct = jnp.transpose(tc).astype(jnp.uint8)
            for u in range(16):
                r_row = tr[u:u + 1, :]
                w_row = tw[u:u + 1, :]
                c_col = tct[:, u:u + 1]
                a_mat = jnp.where(r_row == iota_a, w_row, zero)
                ct_mat = jnp.where(c_col == iota_ct, one8, zero8)
                acc = acc + jax.lax.dot_general(
                    a_mat, ct_mat, (((1,), (0,)), ((), ())),
                    preferred_element_type=jnp.float32)
        return acc

    acc = jax.lax.fori_loop(0, R // 64, body,
                            jnp.zeros((S, 2 * S), jnp.float32))
    o_ref[0] = acc[:, :S]
    o_ref[1] = acc[:, S:]


def kernel(x):
    B, C, N = x.shape
    R = N // 128
    xr = jax.lax.slice(x, (0, 0, 0), (B, 2, N)).reshape(B, 2, R, 128)
    out = pl.pallas_call(
        _splat_kernel,
        grid=(B // 2,),
        in_specs=[pl.BlockSpec((2, 2, R, 128), lambda p: (p, 0, 0, 0))],
        out_specs=pl.BlockSpec((2, S, S), lambda p: (p, 0, 0)),
        out_shape=jax.ShapeDtypeStruct((B, S, S), jnp.float32),
        scratch_shapes=[
            pltpu.VMEM((R, 2 * S), jnp.int16),
            pltpu.VMEM((R, 2 * S), jnp.int32),
            pltpu.VMEM((R, 2 * S), jnp.bfloat16),
        ],
        compiler_params=pltpu.CompilerParams(
            dimension_semantics=("parallel",)),
    )(xr)
    return out[:, None, :, :]


# speedup vs baseline: 10.3840x; 1.0799x over previous
import jax
import jax.numpy as jnp
from jax.experimental import pallas as pl
from jax.experimental.pallas import tpu as pltpu

S = 128
SCALE = float(S // 2 - 2)


def _splat_kernel(x_ref, o_ref, cr_ref, cc_ref, w_ref):
    R = x_ref.shape[2]

    for b in range(2):
        pc = x_ref[b] * SCALE
        clf = jnp.trunc(pc)
        cli = clf.astype(jnp.int32)
        feat = 2.0 - jnp.abs(clf - pc).sum(axis=0)
        c0 = cli[0] - jnp.min(cli[0])
        c1 = cli[1] - jnp.min(cli[1])
        oob = (c0 >= S) | (c1 >= S)
        c0 = jnp.where(c0 >= S, 0, c0)
        c1 = jnp.where(c1 >= S, 0, c1)
        w = jnp.where(oob, 0.0, feat)
        cr_ref[:, b * S:(b + 1) * S] = c0.astype(jnp.int16)
        cc_ref[:, b * S:(b + 1) * S] = c1 + b * S
        w_ref[:, b * S:(b + 1) * S] = w.astype(jnp.bfloat16)

    iota_a = jax.lax.broadcasted_iota(jnp.int16, (S, 2 * S), 0)
    iota_ct = jax.lax.broadcasted_iota(jnp.int32, (2 * S, 2 * S), 1).astype(jnp.uint8)
    one8 = jnp.float8_e4m3fn(1.0)
    zero8 = jnp.float8_e4m3fn(0.0)
    zero = jnp.bfloat16(0.0)

    def body(i, acc):
        for h in range(8):
            base = i * 128 + h * 16
            tr = cr_ref[pl.ds(base, 16), :]
            tc = cc_ref[pl.ds(base, 16), :]
            tw = w_ref[pl.ds(base, 16), :]
            tct = jnp.transpose(tc).astype(jnp.uint8)
            for u in range(16):
                r_row = tr[u:u + 1, :]
                w_row = tw[u:u + 1, :]
                c_col = tct[:, u:u + 1]
                a_mat = jnp.where(r_row == iota_a, w_row, zero)
                ct_mat = jnp.where(c_col == iota_ct, one8, zero8)
                acc = acc + jax.lax.dot_general(
                    a_mat, ct_mat, (((1,), (0,)), ((), ())),
                    preferred_element_type=jnp.float32)
        return acc

    acc = jax.lax.fori_loop(0, R // 128, body,
                            jnp.zeros((S, 2 * S), jnp.float32))
    o_ref[0] = acc[:, :S]
    o_ref[1] = acc[:, S:]


def kernel(x):
    B, C, N = x.shape
    R = N // 128
    xr = jax.lax.slice(x, (0, 0, 0), (B, 2, N)).reshape(B, 2, R, 128)
    out = pl.pallas_call(
        _splat_kernel,
        grid=(B // 2,),
        in_specs=[pl.BlockSpec((2, 2, R, 128), lambda p: (p, 0, 0, 0))],
        out_specs=pl.BlockSpec((2, S, S), lambda p: (p, 0, 0)),
        out_shape=jax.ShapeDtypeStruct((B, S, S), jnp.float32),
        scratch_shapes=[
            pltpu.VMEM((R, 2 * S), jnp.int16),
            pltpu.VMEM((R, 2 * S), jnp.int32),
            pltpu.VMEM((R, 2 * S), jnp.bfloat16),
        ],
        compiler_params=pltpu.CompilerParams(
            dimension_semantics=("parallel",)),
    )(xr)
    return out[:, None, :, :]


# speedup vs baseline: 10.3862x; 1.0002x over previous
import jax
import jax.numpy as jnp
from jax.experimental import pallas as pl
from jax.experimental.pallas import tpu as pltpu

S = 128
SCALE = float(S // 2 - 2)


def _splat_kernel(x_ref, o_ref, cr_ref, cc_ref, w_ref):
    R = x_ref.shape[2]

    for b in range(2):
        pc = x_ref[b] * SCALE
        clf = jnp.trunc(pc)
        cli = clf.astype(jnp.int32)
        feat = 2.0 - jnp.abs(clf - pc).sum(axis=0)
        c0 = cli[0] - jnp.min(cli[0])
        c1 = cli[1] - jnp.min(cli[1])
        oob = (c0 >= S) | (c1 >= S)
        c0 = jnp.where(c0 >= S, 0, c0)
        c1 = jnp.where(c1 >= S, 0, c1)
        w = jnp.where(oob, 0.0, feat)
        cr_ref[:, b * S:(b + 1) * S] = c0.astype(jnp.int16)
        cc_ref[:, b * S:(b + 1) * S] = c1 + b * S
        w_ref[:, b * S:(b + 1) * S] = w.astype(jnp.bfloat16)

    iota_a = jax.lax.broadcasted_iota(jnp.int16, (S, 2 * S), 0)
    iota_ct = jax.lax.broadcasted_iota(jnp.int32, (2 * S, 2 * S), 1).astype(jnp.uint8)
    one8 = jnp.float8_e4m3fn(1.0)
    zero8 = jnp.float8_e4m3fn(0.0)
    zero = jnp.bfloat16(0.0)

    def body(i, acc):
        for h in range(8):
            base = i * 128 + h * 16
            tr = cr_ref[pl.ds(base, 16), :]
            tc = cc_ref[pl.ds(base, 16), :]
            tw = w_ref[pl.ds(base, 16), :]
            tct = jnp.transpose(tc).astype(jnp.uint8)
            for u in range(16):
                r_row = tr[u:u + 1, :]
                w_row = tw[u:u + 1, :]
                c_col = tct[:, u:u + 1]
                a_mat = jnp.where(r_row == iota_a, w_row, zero)
                ct_mat = jnp.where(c_col == iota_ct, one8, zero8)
                acc = acc + jax.lax.dot_general(
                    a_mat, ct_mat, (((1,), (0,)), ((), ())),
                    preferred_element_type=jnp.float32)
        return acc

    acc = jax.lax.fori_loop(0, R // 128, body,
                            jnp.zeros((S, 2 * S), jnp.float32))
    o_ref[0] = acc[:, :S]
    o_ref[1] = acc[:, S:]


def kernel(x):
    B, C, N = x.shape
    R = N // 128
    xr = jax.lax.slice(x, (0, 0, 0), (B, 2, N)).reshape(B, 2, R, 128)
    out = pl.pallas_call(
        _splat_kernel,
        grid=(B // 2,),
        in_specs=[pl.BlockSpec((2, 2, R, 128), lambda p: (p, 0, 0, 0))],
        out_specs=pl.BlockSpec((2, S, S), lambda p: (p, 0, 0)),
        out_shape=jax.ShapeDtypeStruct((B, S, S), jnp.float32),
        scratch_shapes=[
            pltpu.VMEM((R, 2 * S), jnp.int16),
            pltpu.VMEM((R, 2 * S), jnp.int32),
            pltpu.VMEM((R, 2 * S), jnp.bfloat16),
        ],
        compiler_params=pltpu.CompilerParams(
            dimension_semantics=("parallel",),
            allow_input_fusion=[True]),
    )(xr)
    return out[:, None, :, :]
